# baseline (speedup 1.0000x reference)
import functools
import math

import jax
import jax.numpy as jnp
from jax import lax
from jax.experimental import pallas as pl
from jax.experimental.pallas import tpu as pltpu
from jax.experimental.pallas import tpu_sc as plsc

D = 64
NW = 32
CHUNK = 128
NG = 4
NS = 2
SCALE = 1.0 / math.sqrt(D)


def _make_emb_kernel(b_total: int):
    b_per_w = b_total // NW
    n_chunks = b_per_w // CHUNK
    mesh = plsc.VectorSubcoreMesh(core_axis_name="c", subcore_axis_name="s")

    @functools.partial(
        pl.kernel,
        out_type=jax.ShapeDtypeStruct((b_total, D), jnp.float32),
        mesh=mesh,
        scratch_types=[
            pltpu.VMEM((n_chunks, CHUNK), jnp.int32),
            [pltpu.VMEM((CHUNK, 2 * D), jnp.float32) for _ in range(NG)],
            [pltpu.VMEM((CHUNK, D), jnp.float32) for _ in range(NS)],
            [pltpu.SemaphoreType.DMA for _ in range(NG)],
            [pltpu.SemaphoreType.DMA for _ in range(NS)],
        ],
    )
    def emb(x_hbm, table_hbm, out_hbm, idx_all, rows, srows, gsems, osems):
        wid = lax.axis_index("s") * 2 + lax.axis_index("c")
        base = wid * b_per_w

        pltpu.sync_copy(x_hbm.at[wid], idx_all)

        for g in range(NG):
            pltpu.async_copy(table_hbm.at[idx_all.at[g]], rows[g], gsems[g])

        def chunk_group(i0):
            for k in range(NG):
                i = i0 + k
                g = k % NG
                s = k % NS
                pltpu.make_async_copy(
                    table_hbm.at[idx_all.at[i]], rows[g], gsems[g]
                ).wait()

                @pl.when(i >= NS)
                def _():
                    pltpu.make_async_copy(
                        srows[s], out_hbm.at[pl.ds(0, CHUNK)], osems[s]
                    ).wait()

                def scale_row(r):
                    for c in range(0, D, 16):
                        srows[s][r, pl.ds(c, 16)] = (
                            rows[g][r, pl.ds(c, 16)] * SCALE
                        )

                plsc.parallel_loop(0, CHUNK, unroll=2)(scale_row)

                @pl.when(i + NG < n_chunks)
                def _():
                    pltpu.async_copy(
                        table_hbm.at[idx_all.at[i + NG]], rows[g], gsems[g]
                    )

                pltpu.async_copy(
                    srows[s],
                    out_hbm.at[pl.ds(base + i * CHUNK, CHUNK)],
                    osems[s],
                )

        pl.loop(0, n_chunks, step=NG)(chunk_group)

        for s in range(NS):
            pltpu.make_async_copy(
                srows[s], out_hbm.at[pl.ds(0, CHUNK)], osems[s]
            ).wait()

    return emb


def _make_table_stage(v: int):
    bv = 8192
    grid = (pl.cdiv(v, bv),)

    def body(in_ref, out_ref):
        d_iota = jax.lax.broadcasted_iota(jnp.int32, (D, 2 * D), 0)
        j_iota = jax.lax.broadcasted_iota(jnp.int32, (D, 2 * D), 1)
        eye2 = (d_iota == (j_iota % D)).astype(jnp.float32)
        out_ref[...] = jax.lax.dot_general(
            in_ref[...], eye2, (((0,), (0,)), ((), ())),
            preferred_element_type=jnp.float32,
            precision=jax.lax.Precision.HIGHEST,
        )

    return pl.pallas_call(
        body,
        grid=grid,
        in_specs=[pl.BlockSpec((D, bv), lambda i: (0, i))],
        out_specs=pl.BlockSpec((bv, 2 * D), lambda i: (i, 0)),
        out_shape=jax.ShapeDtypeStruct((v, 2 * D), jnp.float32),
        compiler_params=pltpu.CompilerParams(
            dimension_semantics=("arbitrary",),
        ),
    )


def kernel(x, table):
    nb, s = x.shape
    b_total = nb * s
    x_grouped = x.reshape(NW, (b_total // NW) // CHUNK, CHUNK).astype(jnp.int32)
    table_p = _make_table_stage(table.shape[0])(table.T)
    out = _make_emb_kernel(b_total)(x_grouped, table_p)
    return out.reshape(nb, s, D)


# speedup vs baseline: 1.1047x; 1.1047x over previous
import functools
import math

import jax
import jax.numpy as jnp
from jax import lax
from jax.experimental import pallas as pl
from jax.experimental.pallas import tpu as pltpu
from jax.experimental.pallas import tpu_sc as plsc

D = 64
NW = 32
CHUNK = 128
NG = 4
NS = 2
SCALE = 1.0 / math.sqrt(D)


def _make_emb_kernel(b_total: int):
    b_per_w = b_total // NW
    n_chunks = b_per_w // CHUNK
    mesh = plsc.VectorSubcoreMesh(core_axis_name="c", subcore_axis_name="s")

    @functools.partial(
        pl.kernel,
        out_type=jax.ShapeDtypeStruct((b_total, D), jnp.float32),
        mesh=mesh,
        scratch_types=[
            pltpu.VMEM((n_chunks, CHUNK), jnp.int32),
            [pltpu.VMEM((CHUNK, 2 * D), jnp.float32) for _ in range(NG)],
            [pltpu.VMEM((CHUNK, D), jnp.float32) for _ in range(NS)],
            [pltpu.SemaphoreType.DMA for _ in range(NG)],
            [pltpu.SemaphoreType.DMA for _ in range(NS)],
        ],
    )
    def emb(x_hbm, table_hbm, out_hbm, idx_all, rows, srows, gsems, osems):
        wid = lax.axis_index("s") * 2 + lax.axis_index("c")
        base = wid * b_per_w

        pltpu.sync_copy(x_hbm.at[wid], idx_all)

        for g in range(NG):
            pltpu.async_copy(table_hbm.at[idx_all.at[g]], rows[g], gsems[g])

        def chunk_group(i0):
            for k in range(NG):
                i = i0 + k
                g = k % NG
                s = k % NS
                pltpu.make_async_copy(
                    table_hbm.at[idx_all.at[i]], rows[g], gsems[g]
                ).wait()

                @pl.when(i >= NS)
                def _():
                    pltpu.make_async_copy(
                        srows[s], out_hbm.at[pl.ds(0, CHUNK)], osems[s]
                    ).wait()

                def scale_row(r):
                    for c in range(0, D, 16):
                        srows[s][r, pl.ds(c, 16)] = (
                            rows[g][r, pl.ds(c, 16)] * SCALE
                        )

                plsc.parallel_loop(0, CHUNK, unroll=2)(scale_row)

                @pl.when(i + NG < n_chunks)
                def _():
                    pltpu.async_copy(
                        table_hbm.at[idx_all.at[i + NG]], rows[g], gsems[g]
                    )

                pltpu.async_copy(
                    srows[s],
                    out_hbm.at[pl.ds(base + i * CHUNK, CHUNK)],
                    osems[s],
                )

        pl.loop(0, n_chunks, step=NG)(chunk_group)

        for s in range(NS):
            pltpu.make_async_copy(
                srows[s], out_hbm.at[pl.ds(0, CHUNK)], osems[s]
            ).wait()

    return emb


def _make_table_stage(v: int):
    bv = 4096
    grid = (pl.cdiv(v, bv),)

    def body(in_ref, out_ref):
        d_iota = jax.lax.broadcasted_iota(jnp.int32, (D, 2 * D), 0)
        j_iota = jax.lax.broadcasted_iota(jnp.int32, (D, 2 * D), 1)
        eye2 = (d_iota == (j_iota % D)).astype(jnp.float32)
        out_ref[...] = jax.lax.dot_general(
            in_ref[...], eye2, (((0,), (0,)), ((), ())),
            preferred_element_type=jnp.float32,
        )

    return pl.pallas_call(
        body,
        grid=grid,
        in_specs=[pl.BlockSpec((D, bv), lambda i: (0, i))],
        out_specs=pl.BlockSpec((bv, 2 * D), lambda i: (i, 0)),
        out_shape=jax.ShapeDtypeStruct((v, 2 * D), jnp.float32),
        compiler_params=pltpu.CompilerParams(
            dimension_semantics=("arbitrary",),
        ),
    )


def kernel(x, table):
    nb, s = x.shape
    b_total = nb * s
    x_grouped = x.reshape(NW, (b_total // NW) // CHUNK, CHUNK).astype(jnp.int32)
    table_p = _make_table_stage(table.shape[0])(table.T)
    out = _make_emb_kernel(b_total)(x_grouped, table_p)
    return out.reshape(nb, s, D)


# speedup vs baseline: 1.2148x; 1.0996x over previous
import functools
import math

import jax
import jax.numpy as jnp
from jax import lax
from jax.experimental import pallas as pl
from jax.experimental.pallas import tpu as pltpu
from jax.experimental.pallas import tpu_sc as plsc

D = 64
NW = 32
CHUNK = 128
NG = 4
NS = 2
SCALE = 1.0 / math.sqrt(D)


def _make_emb_kernel(b_total: int):
    b_per_w = b_total // NW
    n_chunks = b_per_w // CHUNK
    mesh = plsc.VectorSubcoreMesh(core_axis_name="c", subcore_axis_name="s")

    @functools.partial(
        pl.kernel,
        out_type=jax.ShapeDtypeStruct((b_total, D), jnp.float32),
        mesh=mesh,
        scratch_types=[
            pltpu.VMEM((n_chunks, CHUNK), jnp.int32),
            [pltpu.VMEM((CHUNK, 2 * D), jnp.float32) for _ in range(NG)],
            [pltpu.VMEM((CHUNK, D), jnp.float32) for _ in range(NS)],
            [pltpu.SemaphoreType.DMA for _ in range(NG)],
            [pltpu.SemaphoreType.DMA for _ in range(NS)],
        ],
    )
    def emb(x_hbm, table_hbm, out_hbm, idx_all, rows, srows, gsems, osems):
        wid = lax.axis_index("s") * 2 + lax.axis_index("c")
        base = wid * b_per_w

        pltpu.sync_copy(x_hbm.at[wid], idx_all)

        for g in range(NG):
            pltpu.async_copy(table_hbm.at[idx_all.at[g]], rows[g], gsems[g])

        def chunk_group(i0):
            for k in range(NG):
                i = i0 + k
                g = k % NG
                s = k % NS
                pltpu.make_async_copy(
                    table_hbm.at[idx_all.at[i]], rows[g], gsems[g]
                ).wait()

                @pl.when(i >= NS)
                def _():
                    pltpu.make_async_copy(
                        srows[s], out_hbm.at[pl.ds(0, CHUNK)], osems[s]
                    ).wait()

                def scale_row(r):
                    for c in range(0, D, 16):
                        srows[s][r, pl.ds(c, 16)] = (
                            rows[g][r, pl.ds(c, 16)] * SCALE
                        )

                plsc.parallel_loop(0, CHUNK, unroll=4)(scale_row)

                @pl.when(i + NG < n_chunks)
                def _():
                    pltpu.async_copy(
                        table_hbm.at[idx_all.at[i + NG]], rows[g], gsems[g]
                    )

                pltpu.async_copy(
                    srows[s],
                    out_hbm.at[pl.ds(base + i * CHUNK, CHUNK)],
                    osems[s],
                )

        pl.loop(0, n_chunks, step=NG)(chunk_group)

        for s in range(NS):
            pltpu.make_async_copy(
                srows[s], out_hbm.at[pl.ds(0, CHUNK)], osems[s]
            ).wait()

    return emb


def _make_table_stage(v: int):
    bv = 8192
    grid = (pl.cdiv(v, bv),)

    def body(in_ref, out_ref):
        d_iota = jax.lax.broadcasted_iota(jnp.int32, (D, 2 * D), 0)
        j_iota = jax.lax.broadcasted_iota(jnp.int32, (D, 2 * D), 1)
        eye2 = (d_iota == (j_iota % D)).astype(jnp.float32)
        out_ref[...] = jax.lax.dot_general(
            in_ref[...], eye2, (((0,), (0,)), ((), ())),
            preferred_element_type=jnp.float32,
        )

    return pl.pallas_call(
        body,
        grid=grid,
        in_specs=[pl.BlockSpec((D, bv), lambda i: (0, i))],
        out_specs=pl.BlockSpec((bv, 2 * D), lambda i: (i, 0)),
        out_shape=jax.ShapeDtypeStruct((v, 2 * D), jnp.float32),
        compiler_params=pltpu.CompilerParams(
            dimension_semantics=("arbitrary",),
        ),
    )


def kernel(x, table):
    nb, s = x.shape
    b_total = nb * s
    x_grouped = x.reshape(NW, (b_total // NW) // CHUNK, CHUNK).astype(jnp.int32)
    table_p = _make_table_stage(table.shape[0])(table.T)
    out = _make_emb_kernel(b_total)(x_grouped, table_p)
    return out.reshape(nb, s, D)


# speedup vs baseline: 1.2559x; 1.0338x over previous
import functools
import math

import jax
import jax.numpy as jnp
from jax import lax
from jax.experimental import pallas as pl
from jax.experimental.pallas import tpu as pltpu
from jax.experimental.pallas import tpu_sc as plsc

D = 64
NW = 32
CHUNK = 128
NG = 4
NS = 2
SCALE = 1.0 / math.sqrt(D)


def _make_emb_kernel(b_total: int):
    b_per_w = b_total // NW
    n_chunks = b_per_w // CHUNK
    mesh = plsc.VectorSubcoreMesh(core_axis_name="c", subcore_axis_name="s")

    @functools.partial(
        pl.kernel,
        out_type=jax.ShapeDtypeStruct((b_total, D), jnp.float32),
        mesh=mesh,
        scratch_types=[
            pltpu.VMEM((n_chunks, CHUNK), jnp.int32),
            [pltpu.VMEM((CHUNK, 2 * D), jnp.float32) for _ in range(NG)],
            [pltpu.VMEM((CHUNK, D), jnp.float32) for _ in range(NS)],
            [pltpu.SemaphoreType.DMA for _ in range(NG)],
            [pltpu.SemaphoreType.DMA for _ in range(NS)],
        ],
    )
    def emb(x_hbm, table_hbm, out_hbm, idx_all, rows, srows, gsems, osems):
        wid = lax.axis_index("s") * 2 + lax.axis_index("c")
        base = wid * b_per_w

        pltpu.sync_copy(x_hbm.at[wid], idx_all)

        for g in range(NG):
            pltpu.async_copy(table_hbm.at[idx_all.at[g]], rows[g], gsems[g])

        def chunk_group(i0):
            for k in range(NG):
                i = i0 + k
                g = k % NG
                s = k % NS
                pltpu.make_async_copy(
                    table_hbm.at[idx_all.at[i]], rows[g], gsems[g]
                ).wait()

                @pl.when(i >= NS)
                def _():
                    pltpu.make_async_copy(
                        srows[s], out_hbm.at[pl.ds(0, CHUNK)], osems[s]
                    ).wait()

                def scale_row(r):
                    for c in range(0, D, 16):
                        srows[s][r, pl.ds(c, 16)] = (
                            rows[g][r, pl.ds(c, 16)] * SCALE
                        )

                plsc.parallel_loop(0, CHUNK, unroll=4)(scale_row)

                @pl.when(i + NG < n_chunks)
                def _():
                    pltpu.async_copy(
                        table_hbm.at[idx_all.at[i + NG]], rows[g], gsems[g]
                    )

                pltpu.async_copy(
                    srows[s],
                    out_hbm.at[pl.ds(base + i * CHUNK, CHUNK)],
                    osems[s],
                )

        pl.loop(0, n_chunks, step=NG)(chunk_group)

        for s in range(NS):
            pltpu.make_async_copy(
                srows[s], out_hbm.at[pl.ds(0, CHUNK)], osems[s]
            ).wait()

    return emb


def _make_table_stage(v: int):
    bv = 16384
    grid = (pl.cdiv(v, bv),)

    def body(in_ref, out_ref):
        d_iota = jax.lax.broadcasted_iota(jnp.int32, (D, 2 * D), 0)
        j_iota = jax.lax.broadcasted_iota(jnp.int32, (D, 2 * D), 1)
        eye2 = (d_iota == (j_iota % D)).astype(jnp.float32)
        out_ref[...] = jax.lax.dot_general(
            in_ref[...], eye2, (((0,), (0,)), ((), ())),
            preferred_element_type=jnp.float32,
        )

    return pl.pallas_call(
        body,
        grid=grid,
        in_specs=[pl.BlockSpec((D, bv), lambda i: (0, i))],
        out_specs=pl.BlockSpec((bv, 2 * D), lambda i: (i, 0)),
        out_shape=jax.ShapeDtypeStruct((v, 2 * D), jnp.float32),
        compiler_params=pltpu.CompilerParams(
            dimension_semantics=("arbitrary",),
        ),
    )


def kernel(x, table):
    nb, s = x.shape
    b_total = nb * s
    x_grouped = x.reshape(NW, (b_total // NW) // CHUNK, CHUNK).astype(jnp.int32)
    table_p = _make_table_stage(table.shape[0])(table.T)
    out = _make_emb_kernel(b_total)(x_grouped, table_p)
    return out.reshape(nb, s, D)


# speedup vs baseline: 1.2672x; 1.0090x over previous
import functools
import math

import jax
import jax.numpy as jnp
from jax import lax
from jax.experimental import pallas as pl
from jax.experimental.pallas import tpu as pltpu
from jax.experimental.pallas import tpu_sc as plsc

D = 64
NW = 32
CHUNK = 128
NG = 4
NS = 2
SCALE = 1.0 / math.sqrt(D)


def _make_emb_kernel(b_total: int):
    b_per_w = b_total // NW
    n_chunks = b_per_w // CHUNK
    mesh = plsc.VectorSubcoreMesh(core_axis_name="c", subcore_axis_name="s")

    @functools.partial(
        pl.kernel,
        out_type=jax.ShapeDtypeStruct((b_total, D), jnp.float32),
        mesh=mesh,
        scratch_types=[
            pltpu.VMEM((n_chunks, CHUNK), jnp.int32),
            [pltpu.VMEM((CHUNK, 2 * D), jnp.float32) for _ in range(NG)],
            [pltpu.VMEM((CHUNK, D), jnp.float32) for _ in range(NS)],
            [pltpu.SemaphoreType.DMA for _ in range(NG)],
            [pltpu.SemaphoreType.DMA for _ in range(NS)],
        ],
    )
    def emb(x_hbm, table_hbm, out_hbm, idx_all, rows, srows, gsems, osems):
        wid = lax.axis_index("s") * 2 + lax.axis_index("c")
        base = wid * b_per_w

        pltpu.sync_copy(x_hbm.at[wid], idx_all)

        for g in range(NG):
            pltpu.async_copy(table_hbm.at[idx_all.at[g]], rows[g], gsems[g])

        def chunk_group(i0):
            for k in range(NG):
                i = i0 + k
                g = k % NG
                s = k % NS
                pltpu.make_async_copy(
                    table_hbm.at[idx_all.at[i]], rows[g], gsems[g]
                ).wait()

                @pl.when(i >= NS)
                def _():
                    pltpu.make_async_copy(
                        srows[s], out_hbm.at[pl.ds(0, CHUNK)], osems[s]
                    ).wait()

                def scale_row(r):
                    for c in range(0, D, 16):
                        srows[s][r, pl.ds(c, 16)] = (
                            rows[g][r, pl.ds(c, 16)] * SCALE
                        )

                plsc.parallel_loop(0, CHUNK, unroll=4)(scale_row)

                @pl.when(i + NG < n_chunks)
                def _():
                    pltpu.async_copy(
                        table_hbm.at[idx_all.at[i + NG]], rows[g], gsems[g]
                    )

                pltpu.async_copy(
                    srows[s],
                    out_hbm.at[pl.ds(base + i * CHUNK, CHUNK)],
                    osems[s],
                )

        pl.loop(0, n_chunks, step=NG)(chunk_group)

        for s in range(NS):
            pltpu.make_async_copy(
                srows[s], out_hbm.at[pl.ds(0, CHUNK)], osems[s]
            ).wait()

    return emb


def _make_table_stage(v: int):
    bv = 32768
    grid = (pl.cdiv(v, bv),)

    def body(in_ref, out_ref):
        d_iota = jax.lax.broadcasted_iota(jnp.int32, (D, 2 * D), 0)
        j_iota = jax.lax.broadcasted_iota(jnp.int32, (D, 2 * D), 1)
        eye2 = (d_iota == (j_iota % D)).astype(jnp.float32)
        out_ref[...] = jax.lax.dot_general(
            in_ref[...], eye2, (((0,), (0,)), ((), ())),
            preferred_element_type=jnp.float32,
        )

    return pl.pallas_call(
        body,
        grid=grid,
        in_specs=[pl.BlockSpec((D, bv), lambda i: (0, i))],
        out_specs=pl.BlockSpec((bv, 2 * D), lambda i: (i, 0)),
        out_shape=jax.ShapeDtypeStruct((v, 2 * D), jnp.float32),
        compiler_params=pltpu.CompilerParams(
            dimension_semantics=("arbitrary",),
        ),
    )


def kernel(x, table):
    nb, s = x.shape
    b_total = nb * s
    x_grouped = x.reshape(NW, (b_total // NW) // CHUNK, CHUNK).astype(jnp.int32)
    table_p = _make_table_stage(table.shape[0])(table.T)
    out = _make_emb_kernel(b_total)(x_grouped, table_p)
    return out.reshape(nb, s, D)
